# baseline (speedup 1.0000x reference)
import functools

import jax
import jax.numpy as jnp
from jax import lax
from jax.experimental import pallas as pl
from jax.experimental.pallas import tpu as pltpu
from jax.experimental.pallas import tpu_sc as plsc

D_MODEL = 1024
NUM_ROWS = 4 * 8192
NUM_WORKERS = 32
ROWS_PER_WORKER = NUM_ROWS // NUM_WORKERS
CHUNK = 16
NUM_CHUNKS = ROWS_PER_WORKER // CHUNK
NBUF = 4
LANES = 16
VREGS_PER_ROW = D_MODEL // LANES
GROUP = 16

_MESH = plsc.VectorSubcoreMesh(core_axis_name="c", subcore_axis_name="s")
_GATHER_DNUMS = lax.GatherDimensionNumbers(
    offset_dims=(), collapsed_slice_dims=(0,), start_index_map=(0,))


@functools.partial(
    pl.kernel,
    mesh=_MESH,
    out_type=jax.ShapeDtypeStruct((NUM_ROWS, D_MODEL), jnp.float32),
    scratch_types=[
        pltpu.VMEM((2, D_MODEL), jnp.float32),
        pltpu.VMEM((ROWS_PER_WORKER,), jnp.int32),
        pltpu.VMEM((CHUNK * LANES,), jnp.int32),
        pltpu.VMEM((CHUNK, D_MODEL), jnp.float32),
        pltpu.VMEM((CHUNK, D_MODEL), jnp.float32),
        pltpu.VMEM((CHUNK, D_MODEL), jnp.float32),
        pltpu.VMEM((CHUNK, D_MODEL), jnp.float32),
        pltpu.SemaphoreType.DMA,
        pltpu.SemaphoreType.DMA,
        pltpu.SemaphoreType.DMA,
        pltpu.SemaphoreType.DMA,
        pltpu.SemaphoreType.DMA,
    ],
)
def _sc_embed(ids_hbm, table_hbm, out_hbm, table_v, ids_v, splat_v,
              buf0, buf1, buf2, buf3, s0, s1, s2, s3, psem):
    wid = lax.axis_index("s") * 2 + lax.axis_index("c")
    base = wid * ROWS_PER_WORKER
    bufs = (buf0, buf1, buf2, buf3)
    ssem = (s0, s1, s2, s3)

    c1 = pltpu.async_copy(table_hbm, table_v, psem)
    c2 = pltpu.async_copy(
        ids_hbm.at[wid // 8, pl.ds((wid % 8) * ROWS_PER_WORKER,
                                   ROWS_PER_WORKER)], ids_v, psem)
    c1.wait()
    c2.wait()

    def build_chunk(chunk_row0, buf):
        def presplat(q, carry):
            ids16 = ids_v[pl.ds(chunk_row0 + q * LANES, LANES)]
            for u in range(LANES):
                splat_v[pl.ds((q * LANES + u) * LANES, LANES)] = lax.gather(
                    ids16, jnp.full((LANES, 1), u, jnp.int32),
                    _GATHER_DNUMS, (1,),
                    mode=lax.GatherScatterMode.PROMISE_IN_BOUNDS)
            return carry

        lax.fori_loop(0, CHUNK // LANES, presplat, 0)

        def grp(g, carry):
            gbase = g * (GROUP * LANES)
            t0 = [table_v[0, pl.ds(gbase + k * LANES, LANES)]
                  for k in range(GROUP)]
            t1 = [table_v[1, pl.ds(gbase + k * LANES, LANES)]
                  for k in range(GROUP)]

            def rowblk(r, carry2):
                idv = splat_v[pl.ds(r * LANES, LANES)]
                m = idv == 0
                for k in range(GROUP):
                    buf[r, pl.ds(gbase + k * LANES, LANES)] = (
                        jnp.where(m, t0[k], t1[k]))
                return carry2

            lax.fori_loop(0, CHUNK, rowblk, 0)
            return carry

        lax.fori_loop(0, VREGS_PER_ROW // GROUP, grp, 0)

    def outer(j, carry):
        for b in range(NBUF):
            i = j * NBUF + b

            @pl.when(j > 0)
            def _drain():
                pltpu.make_async_copy(
                    bufs[b], out_hbm.at[pl.ds(0, CHUNK)], ssem[b]).wait()

            chunk_row0 = i * CHUNK
            build_chunk(chunk_row0, bufs[b])
            pltpu.async_copy(
                bufs[b], out_hbm.at[pl.ds(base + chunk_row0, CHUNK)], ssem[b])
        return carry

    lax.fori_loop(0, NUM_CHUNKS // NBUF, outer, 0)
    for b in range(NBUF):
        pltpu.make_async_copy(
            bufs[b], out_hbm.at[pl.ds(0, CHUNK)], ssem[b]).wait()


def kernel(segment_ids, table):
    out = _sc_embed(segment_ids.astype(jnp.int32), table)
    return out.reshape(segment_ids.shape + (D_MODEL,))


# speedup vs baseline: 1.0245x; 1.0245x over previous
import functools

import jax
import jax.numpy as jnp
from jax import lax
from jax.experimental import pallas as pl
from jax.experimental.pallas import tpu as pltpu
from jax.experimental.pallas import tpu_sc as plsc

D_MODEL = 1024
NUM_ROWS = 4 * 8192
NUM_WORKERS = 32
ROWS_PER_WORKER = NUM_ROWS // NUM_WORKERS
CHUNK = 32
NUM_CHUNKS = ROWS_PER_WORKER // CHUNK
NBUF = 2
LANES = 16
VREGS_PER_ROW = D_MODEL // LANES
GROUP = 16

_MESH = plsc.VectorSubcoreMesh(core_axis_name="c", subcore_axis_name="s")
_GATHER_DNUMS = lax.GatherDimensionNumbers(
    offset_dims=(), collapsed_slice_dims=(0,), start_index_map=(0,))


@functools.partial(
    pl.kernel,
    mesh=_MESH,
    out_type=jax.ShapeDtypeStruct((NUM_ROWS, D_MODEL), jnp.float32),
    scratch_types=[
        pltpu.VMEM((2, D_MODEL), jnp.float32),
        pltpu.VMEM((ROWS_PER_WORKER,), jnp.int32),
        pltpu.VMEM((CHUNK * LANES,), jnp.int32),
        pltpu.VMEM((CHUNK, D_MODEL), jnp.float32),
        pltpu.VMEM((CHUNK, D_MODEL), jnp.float32),
        pltpu.SemaphoreType.DMA,
        pltpu.SemaphoreType.DMA,
        pltpu.SemaphoreType.DMA,
    ],
)
def _sc_embed(ids_hbm, table_hbm, out_hbm, table_v, ids_v, splat_v,
              buf0, buf1, s0, s1, psem):
    wid = lax.axis_index("s") * 2 + lax.axis_index("c")
    base = wid * ROWS_PER_WORKER
    bufs = (buf0, buf1)
    ssem = (s0, s1)

    c1 = pltpu.async_copy(table_hbm, table_v, psem)
    c2 = pltpu.async_copy(
        ids_hbm.at[wid // 8, pl.ds((wid % 8) * ROWS_PER_WORKER,
                                   ROWS_PER_WORKER)], ids_v, psem)
    c1.wait()
    c2.wait()

    def build_chunk(chunk_row0, buf):
        def presplat(q, carry):
            ids16 = ids_v[pl.ds(chunk_row0 + q * LANES, LANES)]
            for u in range(LANES):
                splat_v[pl.ds((q * LANES + u) * LANES, LANES)] = lax.gather(
                    ids16, jnp.full((LANES, 1), u, jnp.int32),
                    _GATHER_DNUMS, (1,),
                    mode=lax.GatherScatterMode.PROMISE_IN_BOUNDS)
            return carry

        lax.fori_loop(0, CHUNK // LANES, presplat, 0)

        def grp(g, carry):
            gbase = g * (GROUP * LANES)
            t0 = [table_v[0, pl.ds(gbase + k * LANES, LANES)]
                  for k in range(GROUP)]
            t1 = [table_v[1, pl.ds(gbase + k * LANES, LANES)]
                  for k in range(GROUP)]

            def rowblk(r, carry2):
                idv = splat_v[pl.ds(r * LANES, LANES)]
                m = idv == 0
                for k in range(GROUP):
                    buf[r, pl.ds(gbase + k * LANES, LANES)] = (
                        jnp.where(m, t0[k], t1[k]))
                return carry2

            lax.fori_loop(0, CHUNK, rowblk, 0)
            return carry

        lax.fori_loop(0, VREGS_PER_ROW // GROUP, grp, 0)

    def outer(j, carry):
        for b in range(NBUF):
            i = j * NBUF + b

            @pl.when(j > 0)
            def _drain():
                pltpu.make_async_copy(
                    bufs[b], out_hbm.at[pl.ds(0, CHUNK)], ssem[b]).wait()

            chunk_row0 = i * CHUNK
            build_chunk(chunk_row0, bufs[b])
            pltpu.async_copy(
                bufs[b], out_hbm.at[pl.ds(base + chunk_row0, CHUNK)], ssem[b])
        return carry

    lax.fori_loop(0, NUM_CHUNKS // NBUF, outer, 0)
    for b in range(NBUF):
        pltpu.make_async_copy(
            bufs[b], out_hbm.at[pl.ds(0, CHUNK)], ssem[b]).wait()


def kernel(segment_ids, table):
    out = _sc_embed(segment_ids.astype(jnp.int32), table)
    return out.reshape(segment_ids.shape + (D_MODEL,))
